# baseline (speedup 1.0000x reference)
import jax
import jax.numpy as jnp
from jax import lax
from jax.experimental import pallas as pl
from jax.experimental.pallas import tpu as pltpu
from jax.experimental.pallas import tpu_sc as plsc

V1 = 100000
D = 64
B = 16384
NC = 2
NS = 16
NW = NC * NS
BPW = B // NW
GCH = 128
L = 16
VB = 128
NPAN = (V1 + VB - 1) // VB


def _merge_body(t1t_ref, t2t_ref, out_ref):
    out_ref[:, :D] = t1t_ref[...].T
    out_ref[:, D:] = t2t_ref[...].T


def _gather_body(idx_hbm, merged_hbm, outw_hbm, idx_v, idxm_v, off_v, buf, sem):
    wid = lax.axis_index("s") * NC + lax.axis_index("c")
    base = wid * BPW

    pltpu.sync_copy(idx_hbm.at[pl.ds(base, BPW)], idx_v)

    for c in range(BPW // L):
        v = idx_v[pl.ds(c * L, L)]
        is2 = v >= V1
        idxm = jnp.where(is2, v - (V1 - 1), v)
        off = jnp.where(is2, D, 0)
        idxm_v[c // (GCH // L), pl.ds((c % (GCH // L)) * L, L)] = idxm
        off_v[pl.ds(c * L, L)] = off

    copies = []
    for k in range(BPW // GCH):
        copies.append(pltpu.async_copy(
            merged_hbm.at[idxm_v.at[k]], buf.at[pl.ds(k * GCH, GCH)], sem))
    for cp in copies:
        cp.wait()

    def combine(ch, carry):
        ov = off_v[pl.ds(ch * L, L)]
        for j in range(L):
            off = ov[j]
            r = ch * L + j
            for g in range(D // L):
                buf[r, pl.ds(g * L, L)] = buf[r, pl.ds(off + g * L, L)]
        return carry

    lax.fori_loop(0, BPW // L, combine, 0)
    pltpu.sync_copy(buf, outw_hbm.at[pl.ds(base, BPW)])


@jax.jit
def _split_embedding(indices, table_one, table_two):
    t1t = table_one.T
    t2t = table_two.T

    merged = pl.pallas_call(
        _merge_body,
        grid=(NPAN,),
        in_specs=[
            pl.BlockSpec((D, VB), lambda i: (0, i)),
            pl.BlockSpec((D, VB), lambda i: (0, i)),
        ],
        out_specs=pl.BlockSpec((VB, 2 * D), lambda i: (i, 0)),
        out_shape=jax.ShapeDtypeStruct((V1, 2 * D), jnp.float32),
    )(t1t, t2t)

    mesh = plsc.VectorSubcoreMesh(
        core_axis_name="c", subcore_axis_name="s",
        num_cores=NC, num_subcores=NS)
    params = pltpu.CompilerParams(
        use_tc_tiling_on_sc=True, needs_layout_passes=False)

    outw = pl.kernel(
        _gather_body,
        out_type=jax.ShapeDtypeStruct((B, 2 * D), jnp.float32),
        mesh=mesh,
        compiler_params=params,
        scratch_types=[
            pltpu.VMEM((BPW,), jnp.int32),
            pltpu.VMEM((BPW // GCH, GCH), jnp.int32),
            pltpu.VMEM((BPW,), jnp.int32),
            pltpu.VMEM((BPW, 2 * D), jnp.float32),
            pltpu.SemaphoreType.DMA,
        ],
    )(indices, merged)

    return outw[:, :D]


def kernel(indices, table_one, table_two):
    return _split_embedding(indices, table_one, table_two)


# speedup vs baseline: 3.7417x; 3.7417x over previous
import jax
import jax.numpy as jnp
from jax import lax
from jax.experimental import pallas as pl
from jax.experimental.pallas import tpu as pltpu
from jax.experimental.pallas import tpu_sc as plsc

V1 = 100000
D = 64
B = 16384
NC = 2
NS = 16
NW = NC * NS
BPW = B // NW
GCH = 128
L = 16
VB = 1024
NPAN = (V1 + VB - 1) // VB


def _merge_body(t1t_ref, t2t_ref, out_ref):
    out_ref[:, :D] = t1t_ref[...].T
    out_ref[:, D:] = t2t_ref[...].T


def _gather_body(idx_hbm, merged_hbm, outw_hbm, idx_v, idxm_v, off_v, buf, sem):
    wid = lax.axis_index("s") * NC + lax.axis_index("c")
    base = wid * BPW

    pltpu.sync_copy(idx_hbm.at[pl.ds(base, BPW)], idx_v)

    for c in range(BPW // L):
        v = idx_v[pl.ds(c * L, L)]
        is2 = v >= V1
        idxm = jnp.where(is2, v - (V1 - 1), v)
        off = jnp.where(is2, D, 0)
        idxm_v[c // (GCH // L), pl.ds((c % (GCH // L)) * L, L)] = idxm
        off_v[pl.ds(c * L, L)] = off

    copies = []
    for k in range(BPW // GCH):
        copies.append(pltpu.async_copy(
            merged_hbm.at[idxm_v.at[k]], buf.at[pl.ds(k * GCH, GCH)], sem))
    for cp in copies:
        cp.wait()

    def combine(ch, carry):
        ov = off_v[pl.ds(ch * L, L)]
        for j in range(L):
            off = ov[j]
            r = ch * L + j
            for g in range(D // L):
                buf[r, pl.ds(g * L, L)] = buf[r, pl.ds(off + g * L, L)]
        return carry

    lax.fori_loop(0, BPW // L, combine, 0)
    pltpu.sync_copy(buf, outw_hbm.at[pl.ds(base, BPW)])


@jax.jit
def _split_embedding(indices, table_one, table_two):
    t1t = table_one.T
    t2t = table_two.T

    merged = pl.pallas_call(
        _merge_body,
        grid=(NPAN,),
        in_specs=[
            pl.BlockSpec((D, VB), lambda i: (0, i)),
            pl.BlockSpec((D, VB), lambda i: (0, i)),
        ],
        out_specs=pl.BlockSpec((VB, 2 * D), lambda i: (i, 0)),
        out_shape=jax.ShapeDtypeStruct((V1, 2 * D), jnp.float32),
    )(t1t, t2t)

    mesh = plsc.VectorSubcoreMesh(
        core_axis_name="c", subcore_axis_name="s",
        num_cores=NC, num_subcores=NS)
    params = pltpu.CompilerParams(
        use_tc_tiling_on_sc=True, needs_layout_passes=False)

    outw = pl.kernel(
        _gather_body,
        out_type=jax.ShapeDtypeStruct((B, 2 * D), jnp.float32),
        mesh=mesh,
        compiler_params=params,
        scratch_types=[
            pltpu.VMEM((BPW,), jnp.int32),
            pltpu.VMEM((BPW // GCH, GCH), jnp.int32),
            pltpu.VMEM((BPW,), jnp.int32),
            pltpu.VMEM((BPW, 2 * D), jnp.float32),
            pltpu.SemaphoreType.DMA,
        ],
    )(indices, merged)

    return outw[:, :D]


def kernel(indices, table_one, table_two):
    return _split_embedding(indices, table_one, table_two)


# speedup vs baseline: 5.2530x; 1.4039x over previous
import jax
import jax.numpy as jnp
from jax import lax
from jax.experimental import pallas as pl
from jax.experimental.pallas import tpu as pltpu
from jax.experimental.pallas import tpu_sc as plsc

V1 = 100000
D = 64
B = 16384
NC = 2
NS = 16
NW = NC * NS
BPW = B // NW
GCH = 128
L = 16
VB = 4096
NPAN = (V1 + VB - 1) // VB


def _merge_body(t1t_ref, t2t_ref, out_ref):
    out_ref[:, :D] = t1t_ref[...].T
    out_ref[:, D:] = t2t_ref[...].T


def _gather_body(idx_hbm, merged_hbm, outw_hbm, idx_v, idxm_v, off_v, buf, sem):
    wid = lax.axis_index("s") * NC + lax.axis_index("c")
    base = wid * BPW

    pltpu.sync_copy(idx_hbm.at[pl.ds(base, BPW)], idx_v)

    for c in range(BPW // L):
        v = idx_v[pl.ds(c * L, L)]
        is2 = v >= V1
        idxm = jnp.where(is2, v - (V1 - 1), v)
        off = jnp.where(is2, D, 0)
        idxm_v[c // (GCH // L), pl.ds((c % (GCH // L)) * L, L)] = idxm
        off_v[pl.ds(c * L, L)] = off

    copies = []
    for k in range(BPW // GCH):
        copies.append(pltpu.async_copy(
            merged_hbm.at[idxm_v.at[k]], buf.at[pl.ds(k * GCH, GCH)], sem))
    for cp in copies:
        cp.wait()

    def combine(ch, carry):
        ov = off_v[pl.ds(ch * L, L)]
        for j in range(L):
            off = ov[j]
            r = ch * L + j
            for g in range(D // L):
                buf[r, pl.ds(g * L, L)] = buf[r, pl.ds(off + g * L, L)]
        return carry

    lax.fori_loop(0, BPW // L, combine, 0)
    pltpu.sync_copy(buf, outw_hbm.at[pl.ds(base, BPW)])


@jax.jit
def _split_embedding(indices, table_one, table_two):
    t1t = table_one.T
    t2t = table_two.T

    merged = pl.pallas_call(
        _merge_body,
        grid=(NPAN,),
        in_specs=[
            pl.BlockSpec((D, VB), lambda i: (0, i)),
            pl.BlockSpec((D, VB), lambda i: (0, i)),
        ],
        out_specs=pl.BlockSpec((VB, 2 * D), lambda i: (i, 0)),
        out_shape=jax.ShapeDtypeStruct((V1, 2 * D), jnp.float32),
    )(t1t, t2t)

    mesh = plsc.VectorSubcoreMesh(
        core_axis_name="c", subcore_axis_name="s",
        num_cores=NC, num_subcores=NS)
    params = pltpu.CompilerParams(
        use_tc_tiling_on_sc=True, needs_layout_passes=False)

    outw = pl.kernel(
        _gather_body,
        out_type=jax.ShapeDtypeStruct((B, 2 * D), jnp.float32),
        mesh=mesh,
        compiler_params=params,
        scratch_types=[
            pltpu.VMEM((BPW,), jnp.int32),
            pltpu.VMEM((BPW // GCH, GCH), jnp.int32),
            pltpu.VMEM((BPW,), jnp.int32),
            pltpu.VMEM((BPW, 2 * D), jnp.float32),
            pltpu.SemaphoreType.DMA,
        ],
    )(indices, merged)

    return outw[:, :D]


def kernel(indices, table_one, table_two):
    return _split_embedding(indices, table_one, table_two)


# speedup vs baseline: 5.6972x; 1.0846x over previous
import jax
import jax.numpy as jnp
from jax import lax
from jax.experimental import pallas as pl
from jax.experimental.pallas import tpu as pltpu
from jax.experimental.pallas import tpu_sc as plsc

V1 = 100000
D = 64
B = 16384
NC = 2
NS = 16
NW = NC * NS
BPW = B // NW
GCH = 128
L = 16
VB = 12544
NPAN = (V1 + VB - 1) // VB


def _merge_body(t1t_ref, t2t_ref, out_ref):
    out_ref[:, :D] = t1t_ref[...].T
    out_ref[:, D:] = t2t_ref[...].T


def _gather_body(idx_hbm, merged_hbm, outw_hbm, idx_v, idxm_v, off_v, buf, sem):
    wid = lax.axis_index("s") * NC + lax.axis_index("c")
    base = wid * BPW

    pltpu.sync_copy(idx_hbm.at[pl.ds(base, BPW)], idx_v)

    for c in range(BPW // L):
        v = idx_v[pl.ds(c * L, L)]
        is2 = v >= V1
        idxm = jnp.where(is2, v - (V1 - 1), v)
        off = jnp.where(is2, D, 0)
        idxm_v[c // (GCH // L), pl.ds((c % (GCH // L)) * L, L)] = idxm
        off_v[pl.ds(c * L, L)] = off

    copies = []
    for k in range(BPW // GCH):
        copies.append(pltpu.async_copy(
            merged_hbm.at[idxm_v.at[k]], buf.at[pl.ds(k * GCH, GCH)], sem))
    for cp in copies:
        cp.wait()

    def combine(ch, carry):
        ov = off_v[pl.ds(ch * L, L)]
        for j in range(L):
            off = ov[j]
            r = ch * L + j
            for g in range(D // L):
                buf[r, pl.ds(g * L, L)] = buf[r, pl.ds(off + g * L, L)]
        return carry

    lax.fori_loop(0, BPW // L, combine, 0)
    pltpu.sync_copy(buf, outw_hbm.at[pl.ds(base, BPW)])


@jax.jit
def _split_embedding(indices, table_one, table_two):
    t1t = table_one.T
    t2t = table_two.T

    merged = pl.pallas_call(
        _merge_body,
        grid=(NPAN,),
        in_specs=[
            pl.BlockSpec((D, VB), lambda i: (0, i)),
            pl.BlockSpec((D, VB), lambda i: (0, i)),
        ],
        out_specs=pl.BlockSpec((VB, 2 * D), lambda i: (i, 0)),
        out_shape=jax.ShapeDtypeStruct((V1, 2 * D), jnp.float32),
    )(t1t, t2t)

    mesh = plsc.VectorSubcoreMesh(
        core_axis_name="c", subcore_axis_name="s",
        num_cores=NC, num_subcores=NS)
    params = pltpu.CompilerParams(
        use_tc_tiling_on_sc=True, needs_layout_passes=False)

    outw = pl.kernel(
        _gather_body,
        out_type=jax.ShapeDtypeStruct((B, 2 * D), jnp.float32),
        mesh=mesh,
        compiler_params=params,
        scratch_types=[
            pltpu.VMEM((BPW,), jnp.int32),
            pltpu.VMEM((BPW // GCH, GCH), jnp.int32),
            pltpu.VMEM((BPW,), jnp.int32),
            pltpu.VMEM((BPW, 2 * D), jnp.float32),
            pltpu.SemaphoreType.DMA,
        ],
    )(indices, merged)

    return outw[:, :D]


def kernel(indices, table_one, table_two):
    return _split_embedding(indices, table_one, table_two)


# speedup vs baseline: 5.7783x; 1.0142x over previous
import jax
import jax.numpy as jnp
from jax import lax
from jax.experimental import pallas as pl
from jax.experimental.pallas import tpu as pltpu
from jax.experimental.pallas import tpu_sc as plsc

V1 = 100000
D = 64
B = 16384
NC = 2
NS = 16
NW = NC * NS
BPW = B // NW
GCH = 128
L = 16
VB = 12544
NPAN = (V1 + VB - 1) // VB


def _merge_body(t1t_ref, t2t_ref, out_ref):
    out_ref[:, :D] = t1t_ref[...].T
    out_ref[:, D:] = t2t_ref[...].T


def _gather_body(idx_hbm, merged_hbm, outw_hbm,
                 idx_v, idxm_v, off_v, buf, sem, osem):
    wid = lax.axis_index("s") * NC + lax.axis_index("c")
    base = wid * BPW

    pltpu.sync_copy(idx_hbm.at[pl.ds(base, BPW)], idx_v)

    for c in range(BPW // L):
        v = idx_v[pl.ds(c * L, L)]
        is2 = v >= V1
        idxm = jnp.where(is2, v - (V1 - 1), v)
        off = jnp.where(is2, D, 0)
        idxm_v[c // (GCH // L), pl.ds((c % (GCH // L)) * L, L)] = idxm
        off_v[pl.ds(c * L, L)] = off

    copies = []
    for k in range(BPW // GCH):
        copies.append(pltpu.async_copy(
            merged_hbm.at[idxm_v.at[k]], buf.at[pl.ds(k * GCH, GCH)], sem))

    def combine(ch, carry):
        ov = off_v[pl.ds(ch * L, L)]
        for j in range(L):
            off = ov[j]
            r = ch * L + j
            for g in range(D // L):
                buf[r, pl.ds(g * L, L)] = buf[r, pl.ds(off + g * L, L)]
        return carry

    outcps = []
    for k in range(BPW // GCH):
        copies[k].wait()
        lax.fori_loop(k * (GCH // L), (k + 1) * (GCH // L), combine, 0)
        cp = pltpu.make_async_copy(
            buf.at[pl.ds(k * GCH, GCH)],
            outw_hbm.at[pl.ds(base + k * GCH, GCH)], osem)
        cp.start()
        outcps.append(cp)
    for cp in outcps:
        cp.wait()


@jax.jit
def _split_embedding(indices, table_one, table_two):
    t1t = table_one.T
    t2t = table_two.T

    merged = pl.pallas_call(
        _merge_body,
        grid=(NPAN,),
        in_specs=[
            pl.BlockSpec((D, VB), lambda i: (0, i)),
            pl.BlockSpec((D, VB), lambda i: (0, i)),
        ],
        out_specs=pl.BlockSpec((VB, 2 * D), lambda i: (i, 0)),
        out_shape=jax.ShapeDtypeStruct((V1, 2 * D), jnp.float32),
    )(t1t, t2t)

    mesh = plsc.VectorSubcoreMesh(
        core_axis_name="c", subcore_axis_name="s",
        num_cores=NC, num_subcores=NS)
    params = pltpu.CompilerParams(
        use_tc_tiling_on_sc=True, needs_layout_passes=False)

    outw = pl.kernel(
        _gather_body,
        out_type=jax.ShapeDtypeStruct((B, 2 * D), jnp.float32),
        mesh=mesh,
        compiler_params=params,
        scratch_types=[
            pltpu.VMEM((BPW,), jnp.int32),
            pltpu.VMEM((BPW // GCH, GCH), jnp.int32),
            pltpu.VMEM((BPW,), jnp.int32),
            pltpu.VMEM((BPW, 2 * D), jnp.float32),
            pltpu.SemaphoreType.DMA,
            pltpu.SemaphoreType.DMA,
        ],
    )(indices, merged)

    return outw[:, :D]


def kernel(indices, table_one, table_two):
    return _split_embedding(indices, table_one, table_two)
